# Initial kernel scaffold; baseline (speedup 1.0000x reference)
import functools

import jax
import jax.numpy as jnp
from jax import lax
from jax.experimental import pallas as pl
from jax.experimental.pallas import tpu as pltpu
from jax.experimental.pallas import tpu_sc as plsc

_G = 256
_EPS = 1e-9
_L = 16
_NC, _NS = 2, 16
_NW = _NC * _NS
_C = 160


def _worker_id():
  return lax.axis_index("s") * _NC + lax.axis_index("c")


def _sc_mesh():
  return plsc.VectorSubcoreMesh(
      core_axis_name="c", subcore_axis_name="s",
      num_cores=_NC, num_subcores=_NS)


def _stats(x, batch, interpret=False):
  n, d = x.shape
  nf = d // _L
  n_chunks = n // _C
  assert n_chunks * _C == n

  @functools.partial(
      pl.kernel,
      out_type=[
          jax.ShapeDtypeStruct((_NW, _G, d), jnp.float32),
          jax.ShapeDtypeStruct((_NW, _G, d), jnp.float32),
          jax.ShapeDtypeStruct((_NW, _G, _L), jnp.float32),
      ],
      mesh=_sc_mesh(),
      scratch_types=[
          pltpu.VMEM((_C, d), jnp.float32),
          pltpu.VMEM((_C,), jnp.int32),
          pltpu.VMEM((_G, d), jnp.float32),
          pltpu.VMEM((_G, d), jnp.float32),
          pltpu.VMEM((_G, _L), jnp.float32),
      ],
      interpret=interpret,
  )
  def k(x_hbm, b_hbm, sum_hbm, sq_hbm, cnt_hbm, xv, iv, sumv, sqv, cntv):
    wid = _worker_id()
    zeros = jnp.zeros((_L,), jnp.float32)
    ones = jnp.ones((_L,), jnp.float32)

    def zero_body(g, carry):
      for f in range(nf):
        s = pl.ds(f * _L, _L)
        sumv[g, s] = zeros
        sqv[g, s] = zeros
      cntv[g, :] = zeros
      return carry

    lax.fori_loop(0, _G, zero_body, 0)

    lo = (n_chunks * wid) // _NW
    hi = (n_chunks * (wid + 1)) // _NW

    def chunk_body(c, carry):
      base = c * _C
      pltpu.sync_copy(x_hbm.at[pl.ds(base, _C)], xv)
      pltpu.sync_copy(b_hbm.at[pl.ds(base, _C)], iv)

      def row_body(r, rc):
        g = iv[r]
        for f in range(nf):
          s = pl.ds(f * _L, _L)
          v = xv[r, s]
          plsc.addupdate(sumv.at[g, s], v)
          plsc.addupdate(sqv.at[g, s], v * v)
        plsc.addupdate(cntv.at[g, :], ones)
        return rc

      lax.fori_loop(0, _C, row_body, 0)
      return carry

    lax.fori_loop(lo, hi, chunk_body, 0)
    pltpu.sync_copy(sumv, sum_hbm.at[wid])
    pltpu.sync_copy(sqv, sq_hbm.at[wid])
    pltpu.sync_copy(cntv, cnt_hbm.at[wid])

  return k(x, batch)


def _combine(sum_p, sq_p, cnt_p, alpha, weight, bias, interpret=False):
  d = sum_p.shape[-1]

  def k(sum_ref, sq_ref, cnt_ref, a_ref, w_ref, b_ref, scale_ref, shift_ref):
    sums = jnp.sum(sum_ref[...], axis=0)
    sqs = jnp.sum(sq_ref[...], axis=0)
    cnt = jnp.sum(cnt_ref[...], axis=0)[:, 0:1]
    cnt = jnp.maximum(cnt, 1.0)
    mu = sums / cnt
    m2 = sqs / cnt
    a = a_ref[...]
    w = w_ref[...]
    b = b_ref[...]
    sig2 = m2 - (2.0 * a - a * a) * mu * mu
    sig2 = jnp.maximum(sig2, 0.0) + _EPS
    rstd = lax.rsqrt(sig2)
    scale_ref[...] = w * rstd
    shift_ref[...] = b - w * a * mu * rstd

  return pl.pallas_call(
      k,
      out_shape=[
          jax.ShapeDtypeStruct((_G, d), jnp.float32),
          jax.ShapeDtypeStruct((_G, d), jnp.float32),
      ],
      interpret=interpret,
  )(sum_p, sq_p, cnt_p, alpha, weight, bias)


def _apply(x, batch, scale, shift, interpret=False):
  n, d = x.shape
  nf = d // _L
  n_chunks = n // _C

  @functools.partial(
      pl.kernel,
      out_type=jax.ShapeDtypeStruct((n, d), jnp.float32),
      mesh=_sc_mesh(),
      scratch_types=[
          pltpu.VMEM((_C, d), jnp.float32),
          pltpu.VMEM((_C,), jnp.int32),
          pltpu.VMEM((_G, d), jnp.float32),
          pltpu.VMEM((_G, d), jnp.float32),
      ],
      interpret=interpret,
  )
  def k(x_hbm, b_hbm, sc_hbm, sh_hbm, out_hbm, xv, iv, scv, shv):
    wid = _worker_id()
    pltpu.sync_copy(sc_hbm, scv)
    pltpu.sync_copy(sh_hbm, shv)

    lo = (n_chunks * wid) // _NW
    hi = (n_chunks * (wid + 1)) // _NW

    def chunk_body(c, carry):
      base = c * _C
      pltpu.sync_copy(x_hbm.at[pl.ds(base, _C)], xv)
      pltpu.sync_copy(b_hbm.at[pl.ds(base, _C)], iv)

      def row_body(r, rc):
        g = iv[r]
        for f in range(nf):
          s = pl.ds(f * _L, _L)
          xv[r, s] = xv[r, s] * scv[g, s] + shv[g, s]
        return rc

      lax.fori_loop(0, _C, row_body, 0)
      pltpu.sync_copy(xv, out_hbm.at[pl.ds(base, _C)])
      return carry

    lax.fori_loop(lo, hi, chunk_body, 0)

  return k(x, batch, scale, shift)


def kernel(x, batch, alpha, weight, bias):
  batch = batch.astype(jnp.int32)
  sum_p, sq_p, cnt_p = _stats(x, batch)
  scale, shift = _combine(
      sum_p, sq_p, cnt_p,
      alpha.reshape(1, -1), weight.reshape(1, -1), bias.reshape(1, -1))
  return _apply(x, batch, scale, shift)


# speedup vs baseline: 2.9633x; 2.9633x over previous
import functools

import jax
import jax.numpy as jnp
from jax import lax
from jax.experimental import pallas as pl
from jax.experimental.pallas import tpu as pltpu
from jax.experimental.pallas import tpu_sc as plsc

_G = 256
_EPS = 1e-9
_L = 16
_NC, _NS = 2, 16
_NW = _NC * _NS
_C = 160


def _worker_id():
  return lax.axis_index("s") * _NC + lax.axis_index("c")


def _sc_mesh():
  return plsc.VectorSubcoreMesh(
      core_axis_name="c", subcore_axis_name="s",
      num_cores=_NC, num_subcores=_NS)


def _stats(x, batch, interpret=False):
  n, d = x.shape
  nf = d // _L
  n_chunks = n // _C
  assert n_chunks * _C == n

  @functools.partial(
      pl.kernel,
      out_type=[
          jax.ShapeDtypeStruct((_NW, _G, d), jnp.float32),
          jax.ShapeDtypeStruct((_NW, _G, d), jnp.float32),
          jax.ShapeDtypeStruct((_NW, _G, _L), jnp.float32),
      ],
      mesh=_sc_mesh(),
      scratch_types=[
          pltpu.VMEM((_C, d), jnp.float32),
          pltpu.VMEM((_C,), jnp.int32),
          pltpu.VMEM((_G, d), jnp.float32),
          pltpu.VMEM((_G, d), jnp.float32),
          pltpu.VMEM((_G, _L), jnp.float32),
      ],
      interpret=interpret,
  )
  def k(x_hbm, b_hbm, sum_hbm, sq_hbm, cnt_hbm, xv, iv, sumv, sqv, cntv):
    wid = _worker_id()
    zeros = jnp.zeros((_L,), jnp.float32)
    ones = jnp.ones((_L,), jnp.float32)

    def zero_body(g, carry):
      for f in range(nf):
        s = pl.ds(f * _L, _L)
        sumv[g, s] = zeros
        sqv[g, s] = zeros
      cntv[g, :] = zeros
      return carry

    lax.fori_loop(0, _G, zero_body, 0)

    lo = (n_chunks * wid) // _NW
    hi = (n_chunks * (wid + 1)) // _NW

    def chunk_body(c, carry):
      base = c * _C
      pltpu.sync_copy(x_hbm.at[pl.ds(base, _C)], xv)
      pltpu.sync_copy(b_hbm.at[pl.ds(base, _C)], iv)

      def grp_body(q, rc):
        gvec = iv[pl.ds(q * _L, _L)]
        for j in range(_L):
          g = gvec[j]
          r = q * _L + j
          for f in range(nf):
            s = pl.ds(f * _L, _L)
            v = xv[r, s]
            plsc.addupdate(sumv.at[g, s], v)
            plsc.addupdate(sqv.at[g, s], v * v)
          plsc.addupdate(cntv.at[g, :], ones)
        return rc

      lax.fori_loop(0, _C // _L, grp_body, 0)
      return carry

    lax.fori_loop(lo, hi, chunk_body, 0)
    pltpu.sync_copy(sumv, sum_hbm.at[wid])
    pltpu.sync_copy(sqv, sq_hbm.at[wid])
    pltpu.sync_copy(cntv, cnt_hbm.at[wid])

  return k(x, batch)


def _combine(sum_p, sq_p, cnt_p, alpha, weight, bias, interpret=False):
  d = sum_p.shape[-1]

  def k(sum_ref, sq_ref, cnt_ref, a_ref, w_ref, b_ref, scale_ref, shift_ref):
    sums = jnp.sum(sum_ref[...], axis=0)
    sqs = jnp.sum(sq_ref[...], axis=0)
    cnt = jnp.sum(cnt_ref[...], axis=0)[:, 0:1]
    cnt = jnp.maximum(cnt, 1.0)
    mu = sums / cnt
    m2 = sqs / cnt
    a = a_ref[...]
    w = w_ref[...]
    b = b_ref[...]
    sig2 = m2 - (2.0 * a - a * a) * mu * mu
    sig2 = jnp.maximum(sig2, 0.0) + _EPS
    rstd = lax.rsqrt(sig2)
    scale_ref[...] = w * rstd
    shift_ref[...] = b - w * a * mu * rstd

  return pl.pallas_call(
      k,
      out_shape=[
          jax.ShapeDtypeStruct((_G, d), jnp.float32),
          jax.ShapeDtypeStruct((_G, d), jnp.float32),
      ],
      interpret=interpret,
  )(sum_p, sq_p, cnt_p, alpha, weight, bias)


def _apply(x, batch, scale, shift, interpret=False):
  n, d = x.shape
  nf = d // _L
  n_chunks = n // _C

  @functools.partial(
      pl.kernel,
      out_type=jax.ShapeDtypeStruct((n, d), jnp.float32),
      mesh=_sc_mesh(),
      scratch_types=[
          pltpu.VMEM((_C, d), jnp.float32),
          pltpu.VMEM((_C,), jnp.int32),
          pltpu.VMEM((_G, d), jnp.float32),
          pltpu.VMEM((_G, d), jnp.float32),
      ],
      interpret=interpret,
  )
  def k(x_hbm, b_hbm, sc_hbm, sh_hbm, out_hbm, xv, iv, scv, shv):
    wid = _worker_id()
    pltpu.sync_copy(sc_hbm, scv)
    pltpu.sync_copy(sh_hbm, shv)

    lo = (n_chunks * wid) // _NW
    hi = (n_chunks * (wid + 1)) // _NW

    def chunk_body(c, carry):
      base = c * _C
      pltpu.sync_copy(x_hbm.at[pl.ds(base, _C)], xv)
      pltpu.sync_copy(b_hbm.at[pl.ds(base, _C)], iv)

      def grp_body(q, rc):
        gvec = iv[pl.ds(q * _L, _L)]
        for j in range(_L):
          g = gvec[j]
          r = q * _L + j
          for f in range(nf):
            s = pl.ds(f * _L, _L)
            xv[r, s] = xv[r, s] * scv[g, s] + shv[g, s]
        return rc

      lax.fori_loop(0, _C // _L, grp_body, 0)
      pltpu.sync_copy(xv, out_hbm.at[pl.ds(base, _C)])
      return carry

    lax.fori_loop(lo, hi, chunk_body, 0)

  return k(x, batch, scale, shift)


def kernel(x, batch, alpha, weight, bias):
  batch = batch.astype(jnp.int32)
  sum_p, sq_p, cnt_p = _stats(x, batch)
  scale, shift = _combine(
      sum_p, sq_p, cnt_p,
      alpha.reshape(1, -1), weight.reshape(1, -1), bias.reshape(1, -1))
  return _apply(x, batch, scale, shift)


# speedup vs baseline: 6.2762x; 2.1180x over previous
import functools

import jax
import jax.numpy as jnp
from jax import lax
from jax.experimental import pallas as pl
from jax.experimental.pallas import tpu as pltpu
from jax.experimental.pallas import tpu_sc as plsc

_G = 256
_EPS = 1e-9
_L = 16
_NC, _NS = 2, 16
_NW = _NC * _NS
_C = 160


def _worker_id():
  return lax.axis_index("s") * _NC + lax.axis_index("c")


def _sc_mesh():
  return plsc.VectorSubcoreMesh(
      core_axis_name="c", subcore_axis_name="s",
      num_cores=_NC, num_subcores=_NS)


def _stats(x, batch, interpret=False):
  n, d = x.shape
  nf = d // _L
  n_chunks = n // _C
  assert n_chunks * _C == n

  @functools.partial(
      pl.kernel,
      out_type=[
          jax.ShapeDtypeStruct((_NW, _G, d), jnp.float32),
          jax.ShapeDtypeStruct((_NW, _G, d), jnp.float32),
          jax.ShapeDtypeStruct((_NW, _G, _L), jnp.float32),
      ],
      mesh=_sc_mesh(),
      scratch_types=[
          pltpu.VMEM((_C, d), jnp.float32),
          pltpu.VMEM((_C,), jnp.int32),
          pltpu.VMEM((_G, d), jnp.float32),
          pltpu.VMEM((_G, d), jnp.float32),
          pltpu.VMEM((_G, _L), jnp.float32),
      ],
      interpret=interpret,
  )
  def k(x_hbm, b_hbm, sum_hbm, sq_hbm, cnt_hbm, xv, iv, sumv, sqv, cntv):
    wid = _worker_id()
    zeros = jnp.zeros((_L,), jnp.float32)
    ones = jnp.ones((_L,), jnp.float32)

    def zero_body(g, carry):
      for f in range(nf):
        s = pl.ds(f * _L, _L)
        sumv[g, s] = zeros
        sqv[g, s] = zeros
      cntv[g, :] = zeros
      return carry

    lax.fori_loop(0, _G, zero_body, 0)

    lo = (n_chunks * wid) // _NW
    hi = (n_chunks * (wid + 1)) // _NW

    def chunk_body(c, carry):
      base = c * _C
      pltpu.sync_copy(x_hbm.at[pl.ds(base, _C)], xv)
      pltpu.sync_copy(b_hbm.at[pl.ds(base, _C)], iv)

      def grp_body(q, rc):
        gvec = iv[pl.ds(q * _L, _L)]
        g0 = gvec[0]
        g15 = gvec[_L - 1]

        @pl.when(g0 == g15)
        def _fast():
          accs = []
          accq = []
          for f in range(nf):
            s = pl.ds(f * _L, _L)
            v = xv[q * _L, s]
            accs.append(v)
            accq.append(v * v)
          for j in range(1, _L):
            r = q * _L + j
            for f in range(nf):
              s = pl.ds(f * _L, _L)
              v = xv[r, s]
              accs[f] = accs[f] + v
              accq[f] = accq[f] + v * v
          for f in range(nf):
            s = pl.ds(f * _L, _L)
            plsc.addupdate(sumv.at[g0, s], accs[f])
            plsc.addupdate(sqv.at[g0, s], accq[f])
          plsc.addupdate(cntv.at[g0, :], ones * float(_L))

        @pl.when(g0 != g15)
        def _slow():
          for j in range(_L):
            g = gvec[j]
            r = q * _L + j
            for f in range(nf):
              s = pl.ds(f * _L, _L)
              v = xv[r, s]
              plsc.addupdate(sumv.at[g, s], v)
              plsc.addupdate(sqv.at[g, s], v * v)
            plsc.addupdate(cntv.at[g, :], ones)

        return rc

      lax.fori_loop(0, _C // _L, grp_body, 0)
      return carry

    lax.fori_loop(lo, hi, chunk_body, 0)
    pltpu.sync_copy(sumv, sum_hbm.at[wid])
    pltpu.sync_copy(sqv, sq_hbm.at[wid])
    pltpu.sync_copy(cntv, cnt_hbm.at[wid])

  return k(x, batch)


def _combine(sum_p, sq_p, cnt_p, alpha, weight, bias, interpret=False):
  d = sum_p.shape[-1]

  def k(sum_ref, sq_ref, cnt_ref, a_ref, w_ref, b_ref, scale_ref, shift_ref):
    sums = jnp.sum(sum_ref[...], axis=0)
    sqs = jnp.sum(sq_ref[...], axis=0)
    cnt = jnp.sum(cnt_ref[...], axis=0)[:, 0:1]
    cnt = jnp.maximum(cnt, 1.0)
    mu = sums / cnt
    m2 = sqs / cnt
    a = a_ref[...]
    w = w_ref[...]
    b = b_ref[...]
    sig2 = m2 - (2.0 * a - a * a) * mu * mu
    sig2 = jnp.maximum(sig2, 0.0) + _EPS
    rstd = lax.rsqrt(sig2)
    scale_ref[...] = w * rstd
    shift_ref[...] = b - w * a * mu * rstd

  return pl.pallas_call(
      k,
      out_shape=[
          jax.ShapeDtypeStruct((_G, d), jnp.float32),
          jax.ShapeDtypeStruct((_G, d), jnp.float32),
      ],
      interpret=interpret,
  )(sum_p, sq_p, cnt_p, alpha, weight, bias)


def _apply(x, batch, scale, shift, interpret=False):
  n, d = x.shape
  nf = d // _L
  n_chunks = n // _C

  @functools.partial(
      pl.kernel,
      out_type=jax.ShapeDtypeStruct((n, d), jnp.float32),
      mesh=_sc_mesh(),
      scratch_types=[
          pltpu.VMEM((_C, d), jnp.float32),
          pltpu.VMEM((_C,), jnp.int32),
          pltpu.VMEM((_G, d), jnp.float32),
          pltpu.VMEM((_G, d), jnp.float32),
      ],
      interpret=interpret,
  )
  def k(x_hbm, b_hbm, sc_hbm, sh_hbm, out_hbm, xv, iv, scv, shv):
    wid = _worker_id()
    pltpu.sync_copy(sc_hbm, scv)
    pltpu.sync_copy(sh_hbm, shv)

    lo = (n_chunks * wid) // _NW
    hi = (n_chunks * (wid + 1)) // _NW

    def chunk_body(c, carry):
      base = c * _C
      pltpu.sync_copy(x_hbm.at[pl.ds(base, _C)], xv)
      pltpu.sync_copy(b_hbm.at[pl.ds(base, _C)], iv)

      def grp_body(q, rc):
        gvec = iv[pl.ds(q * _L, _L)]
        g0 = gvec[0]
        g15 = gvec[_L - 1]

        @pl.when(g0 == g15)
        def _fast():
          scr = []
          shr = []
          for f in range(nf):
            s = pl.ds(f * _L, _L)
            scr.append(scv[g0, s])
            shr.append(shv[g0, s])
          for j in range(_L):
            r = q * _L + j
            for f in range(nf):
              s = pl.ds(f * _L, _L)
              xv[r, s] = xv[r, s] * scr[f] + shr[f]

        @pl.when(g0 != g15)
        def _slow():
          for j in range(_L):
            g = gvec[j]
            r = q * _L + j
            for f in range(nf):
              s = pl.ds(f * _L, _L)
              xv[r, s] = xv[r, s] * scv[g, s] + shv[g, s]

        return rc

      lax.fori_loop(0, _C // _L, grp_body, 0)
      pltpu.sync_copy(xv, out_hbm.at[pl.ds(base, _C)])
      return carry

    lax.fori_loop(lo, hi, chunk_body, 0)

  return k(x, batch, scale, shift)


def kernel(x, batch, alpha, weight, bias):
  batch = batch.astype(jnp.int32)
  sum_p, sq_p, cnt_p = _stats(x, batch)
  scale, shift = _combine(
      sum_p, sq_p, cnt_p,
      alpha.reshape(1, -1), weight.reshape(1, -1), bias.reshape(1, -1))
  return _apply(x, batch, scale, shift)
